# baseline (speedup 1.0000x reference)
import functools

import jax
import jax.numpy as jnp
from jax import lax
from jax.experimental import pallas as pl
from jax.experimental.pallas import tpu as pltpu
from jax.experimental.pallas import tpu_sc as plsc

_N = 10000
_E = 320000
_D = 128
_L = 4


def _mm_body(x_ref, w_ref, b_ref, o_ref):
    o_ref[...] = (
        jnp.dot(x_ref[...], w_ref[...], preferred_element_type=jnp.float32)
        + b_ref[...]
    )


def _matmul(x, w, b, block_rows):
    rows, k = x.shape
    dout = w.shape[1]
    return pl.pallas_call(
        _mm_body,
        grid=(rows // block_rows,),
        in_specs=[
            pl.BlockSpec((block_rows, k), lambda i: (i, 0)),
            pl.BlockSpec((k, dout), lambda i: (0, 0)),
            pl.BlockSpec((1, dout), lambda i: (0, 0)),
        ],
        out_specs=pl.BlockSpec((block_rows, dout), lambda i: (i, 0)),
        out_shape=jax.ShapeDtypeStruct((rows, dout), jnp.float32),
    )(x, w, b.reshape(1, dout))


def _mm_split_body(x_ref, w_ref, b_ref, o_ref):
    o_ref[0] = (
        jnp.dot(x_ref[...], w_ref[0], preferred_element_type=jnp.float32)
        + b_ref[0]
    )


def _matmul_split(x, w, b, block_rows):
    rows, k = x.shape
    wsp = w.reshape(k, 2, 64).transpose(1, 0, 2)
    bsp = b.reshape(2, 1, 64)
    return pl.pallas_call(
        _mm_split_body,
        grid=(rows // block_rows, 2),
        in_specs=[
            pl.BlockSpec((block_rows, k), lambda i, c: (i, 0)),
            pl.BlockSpec((1, k, 64), lambda i, c: (c, 0, 0)),
            pl.BlockSpec((1, 1, 64), lambda i, c: (c, 0, 0)),
        ],
        out_specs=pl.BlockSpec((1, block_rows, 64), lambda i, c: (c, i, 0)),
        out_shape=jax.ShapeDtypeStruct((2, rows, 64), jnp.float32),
    )(x, wsp, bsp)


_CB = 80
_EPT = _E // 16
_NCH = _EPT // _CB
_NPT = _N // 16
_ZR = 125


def _sc_edge(nm8, ce, srci, dsti):
    mesh = plsc.VectorSubcoreMesh(core_axis_name="c", subcore_axis_name="s")
    out_type = [
        jax.ShapeDtypeStruct((2, _E, 64), jnp.float32),
        jax.ShapeDtypeStruct((2, _N, 64), jnp.float32),
        jax.ShapeDtypeStruct((2, _N, 64), jnp.float32),
        jax.ShapeDtypeStruct((2, 16, _D), jnp.float32),
    ]
    scratch_types = [
        pltpu.VMEM((_CB,), jnp.int32),
        pltpu.VMEM((_CB,), jnp.int32),
        pltpu.VMEM((_CB,), jnp.int32),
        pltpu.VMEM((_CB,), jnp.int32),
        pltpu.VMEM((_CB,), jnp.int32),
        pltpu.VMEM((_CB, 64), jnp.float32),
        pltpu.VMEM((_CB, 64), jnp.float32),
        pltpu.VMEM((_CB, 64), jnp.float32),
        pltpu.VMEM((_CB, 64), jnp.float32),
        pltpu.VMEM((_CB, 64), jnp.float32),
        pltpu.VMEM((_CB, 64), jnp.float32),
        pltpu.VMEM((_CB, 64), jnp.float32),
        pltpu.VMEM((128,), jnp.float32),
        pltpu.VMEM((_ZR, 64), jnp.float32),
        pltpu.VMEM_SHARED((_N, 64), jnp.float32),
        pltpu.VMEM_SHARED((_N, 64), jnp.float32),
    ]

    @functools.partial(
        pl.kernel,
        out_type=out_type,
        mesh=mesh,
        scratch_types=scratch_types,
        compiler_params=pltpu.CompilerParams(use_tc_tiling_on_sc=False),
    )
    def k(
        nm_hbm,
        ce_hbm,
        src_hbm,
        dst_hbm,
        enew_hbm,
        num_hbm,
        den_hbm,
        st_hbm,
        srcv,
        dstv,
        bidxv,
        didxv,
        eidxv,
        bhv,
        dhv,
        ehv,
        cev,
        env,
        sgv,
        ncv,
        statv,
        zv,
        nacc,
        dacc,
    ):
        c = lax.axis_index("c")
        s = lax.axis_index("s")
        zero16 = jnp.zeros((16,), jnp.float32)

        @pl.loop(0, _ZR)
        def _(r):
            for k4 in range(4):
                zv[r, pl.ds(k4 * 16, 16)] = zero16

        for k8 in range(8):
            statv[pl.ds(k8 * 16, 16)] = zero16

        for q in range(_NPT // _ZR):
            row0 = s * _NPT + q * _ZR
            pltpu.sync_copy(zv, nacc.at[pl.ds(row0, _ZR)])
            pltpu.sync_copy(zv, dacc.at[pl.ds(row0, _ZR)])
        plsc.subcore_barrier()

        @pl.loop(0, _NCH)
        def _(g):
            base = s * _EPT + g * _CB
            pltpu.sync_copy(src_hbm.at[pl.ds(base, _CB)], srcv)
            pltpu.sync_copy(dst_hbm.at[pl.ds(base, _CB)], dstv)
            for j in range(_CB // 16):
                sl = pl.ds(j * 16, 16)
                s8 = srcv[sl] * 8
                d8 = dstv[sl] * 8
                bidxv[sl] = s8 + (2 + c)
                didxv[sl] = s8 + (4 + c)
                eidxv[sl] = d8 + (6 + c)
            pltpu.sync_copy(nm_hbm.at[bidxv], bhv)
            pltpu.sync_copy(nm_hbm.at[didxv], dhv)
            pltpu.sync_copy(nm_hbm.at[eidxv], ehv)
            pltpu.sync_copy(ce_hbm.at[c].at[pl.ds(base, _CB)], cev)

            @pl.loop(0, _CB)
            def _(r):
                for k4 in range(4):
                    sl = pl.ds(k4 * 16, 16)
                    en = cev[r, sl] + dhv[r, sl] + ehv[r, sl]
                    env[r, sl] = en
                    sg = 1.0 / (1.0 + jnp.exp(-en))
                    sgv[r, sl] = sg
                    ncv[r, sl] = sg * bhv[r, sl]
                    statv[sl] = statv[sl] + en
                    sq = pl.ds(64 + k4 * 16, 16)
                    statv[sq] = statv[sq] + en * en

            pltpu.sync_copy(env, enew_hbm.at[c].at[pl.ds(base, _CB)])
            pltpu.sync_copy(ncv, nacc.at[dstv], add=True)
            pltpu.sync_copy(sgv, dacc.at[dstv], add=True)

        plsc.subcore_barrier()

        @pl.when(s < 15)
        def _():
            row0 = s * 640
            pltpu.sync_copy(
                nacc.at[pl.ds(row0, 640)], num_hbm.at[c].at[pl.ds(row0, 640)]
            )
            pltpu.sync_copy(
                dacc.at[pl.ds(row0, 640)], den_hbm.at[c].at[pl.ds(row0, 640)]
            )

        @pl.when(s == 15)
        def _():
            pltpu.sync_copy(
                nacc.at[pl.ds(9600, 400)], num_hbm.at[c].at[pl.ds(9600, 400)]
            )
            pltpu.sync_copy(
                dacc.at[pl.ds(9600, 400)], den_hbm.at[c].at[pl.ds(9600, 400)]
            )

        pltpu.sync_copy(statv, st_hbm.at[c].at[s])

    return k(nm8, ce, srci, dsti)


def _h_update(h_in, ah, num, den, gamma, beta):
    def body(h_ref, ah_ref, num_ref, den_ref, g_ref, b_ref, o_ref):
        num2 = jnp.concatenate([num_ref[0], num_ref[1]], axis=1)
        den2 = jnp.concatenate([den_ref[0], den_ref[1]], axis=1)
        h_new = ah_ref[...] + num2 / (den2 + 1e-6)
        mu = jnp.mean(h_new, axis=0, keepdims=True)
        var = jnp.mean((h_new - mu) ** 2, axis=0, keepdims=True)
        bn = (h_new - mu) * lax.rsqrt(var + 1e-5) * g_ref[...] + b_ref[...]
        o_ref[...] = h_ref[...] + jnp.maximum(bn, 0.0)

    return pl.pallas_call(
        body,
        in_specs=[
            pl.BlockSpec((_N, _D), lambda: (0, 0)),
            pl.BlockSpec((_N, _D), lambda: (0, 0)),
            pl.BlockSpec((2, _N, 64), lambda: (0, 0, 0)),
            pl.BlockSpec((2, _N, 64), lambda: (0, 0, 0)),
            pl.BlockSpec((1, _D), lambda: (0, 0)),
            pl.BlockSpec((1, _D), lambda: (0, 0)),
        ],
        out_specs=pl.BlockSpec((_N, _D), lambda: (0, 0)),
        out_shape=jax.ShapeDtypeStruct((_N, _D), jnp.float32),
    )(h_in, ah, num, den, gamma.reshape(1, _D), beta.reshape(1, _D))


_BEU = 2000


def _e_update(e_in, e_new, scale, shift):
    def body(e_ref, lo_ref, hi_ref, sc_ref, sh_ref, o_ref):
        en = jnp.concatenate([lo_ref[0], hi_ref[0]], axis=1)
        o_ref[...] = e_ref[...] + jnp.maximum(
            en * sc_ref[...] + sh_ref[...], 0.0
        )

    return pl.pallas_call(
        body,
        grid=(_E // _BEU,),
        in_specs=[
            pl.BlockSpec((_BEU, _D), lambda i: (i, 0)),
            pl.BlockSpec((1, _BEU, 64), lambda i: (0, i, 0)),
            pl.BlockSpec((1, _BEU, 64), lambda i: (1, i, 0)),
            pl.BlockSpec((1, _D), lambda i: (0, 0)),
            pl.BlockSpec((1, _D), lambda i: (0, 0)),
        ],
        out_specs=pl.BlockSpec((_BEU, _D), lambda i: (i, 0)),
        out_shape=jax.ShapeDtypeStruct((_E, _D), jnp.float32),
    )(e_in, e_new, e_new, scale, shift)


def _readout(h4, w1, b1, w2, b2, w3p, b3p):
    def body(h_ref, w1_ref, b1_ref, w2_ref, b2_ref, w3_ref, b3_ref, o_ref):
        y = jnp.mean(h_ref[...], axis=0, keepdims=True)
        y = jnp.broadcast_to(y, (8, _D))
        y = jnp.maximum(
            jnp.dot(y, w1_ref[...], preferred_element_type=jnp.float32)
            + b1_ref[...],
            0.0,
        )
        y = jnp.maximum(
            jnp.dot(y, w2_ref[...], preferred_element_type=jnp.float32)
            + b2_ref[...],
            0.0,
        )
        o_ref[...] = (
            jnp.dot(y, w3_ref[...], preferred_element_type=jnp.float32)
            + b3_ref[...]
        )

    return pl.pallas_call(
        body,
        in_specs=[
            pl.BlockSpec((_N, _D), lambda: (0, 0)),
            pl.BlockSpec((_D, _D), lambda: (0, 0)),
            pl.BlockSpec((1, _D), lambda: (0, 0)),
            pl.BlockSpec((_D, _D), lambda: (0, 0)),
            pl.BlockSpec((1, _D), lambda: (0, 0)),
            pl.BlockSpec((_D, _D), lambda: (0, 0)),
            pl.BlockSpec((1, _D), lambda: (0, 0)),
        ],
        out_specs=pl.BlockSpec((8, _D), lambda: (0, 0)),
        out_shape=jax.ShapeDtypeStruct((8, _D), jnp.float32),
    )(
        h4,
        w1,
        b1.reshape(1, _D),
        w2,
        b2.reshape(1, _D),
        w3p,
        b3p.reshape(1, _D),
    )


def kernel(
    h,
    e,
    edge_index,
    W_emb_h,
    b_emb_h,
    W_emb_e,
    b_emb_e,
    W_A,
    b_A,
    W_B,
    b_B,
    W_C,
    b_C,
    W_D,
    b_D,
    W_E,
    b_E,
    gamma_h,
    beta_h,
    gamma_e,
    beta_e,
    W1,
    b1,
    W2,
    b2,
    W3,
    b3,
):
    src = edge_index[0].astype(jnp.int32)
    dst = edge_index[1].astype(jnp.int32)

    h = _matmul(h, W_emb_h, b_emb_h, 2000)
    e = _matmul(e, W_emb_e, b_emb_e, 2000)

    for l in range(_L):
        wcat = jnp.concatenate([W_A[l], W_B[l], W_D[l], W_E[l]], axis=1)
        bcat = jnp.concatenate([b_A[l], b_B[l], b_D[l], b_E[l]], axis=0)
        nm = _matmul(h, wcat, bcat, 2000)
        ce = _matmul_split(e, W_C[l], b_C[l], 2000)
        e_new, num, den, st = _sc_edge(nm.reshape(8 * _N, 64), ce, src, dst)
        ah = lax.slice(nm, (0, 0), (_N, _D))
        h = _h_update(h, ah, num, den, gamma_h[l], beta_h[l])
        if l < _L - 1:
            cnt = float(_E)
            ssum = jnp.concatenate(
                [st[0, :, :64].sum(axis=0), st[1, :, :64].sum(axis=0)]
            )
            ssq = jnp.concatenate(
                [st[0, :, 64:].sum(axis=0), st[1, :, 64:].sum(axis=0)]
            )
            mu = ssum / cnt
            var = ssq / cnt - mu * mu
            rstd = lax.rsqrt(var + 1e-5)
            scale = (gamma_e[l] * rstd).reshape(1, _D)
            shift = (beta_e[l] - mu * rstd * gamma_e[l]).reshape(1, _D)
            e = _e_update(e, e_new, scale, shift)

    w3p = jnp.zeros((_D, _D), jnp.float32).at[:, :10].set(W3)
    b3p = jnp.zeros((_D,), jnp.float32).at[:10].set(b3)
    y = _readout(h, W1, b1, W2, b2, w3p, b3p)
    return y[0:1, 0:10]


# speedup vs baseline: 1.2126x; 1.2126x over previous
import functools

import jax
import jax.numpy as jnp
from jax import lax
from jax.experimental import pallas as pl
from jax.experimental.pallas import tpu as pltpu
from jax.experimental.pallas import tpu_sc as plsc

_N = 10000
_E = 320000
_D = 128
_L = 4


def _mm_body(x_ref, w_ref, b_ref, o_ref):
    o_ref[...] = (
        jnp.dot(x_ref[...], w_ref[...], preferred_element_type=jnp.float32)
        + b_ref[...]
    )


def _matmul(x, w, b, block_rows):
    rows, k = x.shape
    dout = w.shape[1]
    return pl.pallas_call(
        _mm_body,
        grid=(rows // block_rows,),
        in_specs=[
            pl.BlockSpec((block_rows, k), lambda i: (i, 0)),
            pl.BlockSpec((k, dout), lambda i: (0, 0)),
            pl.BlockSpec((1, dout), lambda i: (0, 0)),
        ],
        out_specs=pl.BlockSpec((block_rows, dout), lambda i: (i, 0)),
        out_shape=jax.ShapeDtypeStruct((rows, dout), jnp.float32),
    )(x, w, b.reshape(1, dout))


def _mm_split_body(x_ref, w_ref, b_ref, o_ref):
    o_ref[0] = (
        jnp.dot(x_ref[...], w_ref[0], preferred_element_type=jnp.float32)
        + b_ref[0]
    )


def _matmul_split(x, w, b, block_rows):
    rows, k = x.shape
    wsp = w.reshape(k, 2, 64).transpose(1, 0, 2)
    bsp = b.reshape(2, 1, 64)
    return pl.pallas_call(
        _mm_split_body,
        grid=(rows // block_rows, 2),
        in_specs=[
            pl.BlockSpec((block_rows, k), lambda i, c: (i, 0)),
            pl.BlockSpec((1, k, 64), lambda i, c: (c, 0, 0)),
            pl.BlockSpec((1, 1, 64), lambda i, c: (c, 0, 0)),
        ],
        out_specs=pl.BlockSpec((1, block_rows, 64), lambda i, c: (c, i, 0)),
        out_shape=jax.ShapeDtypeStruct((2, rows, 64), jnp.float32),
    )(x, wsp, bsp)


_CB = 80
_EPT = _E // 16
_NCH = _EPT // _CB


def _sc_edge(nm8, ce, sdi):
    mesh = plsc.VectorSubcoreMesh(core_axis_name="c", subcore_axis_name="s")
    out_type = [
        jax.ShapeDtypeStruct((2, _E, 64), jnp.float32),
        jax.ShapeDtypeStruct((2, _N, 64), jnp.float32),
        jax.ShapeDtypeStruct((2, _N, 64), jnp.float32),
        jax.ShapeDtypeStruct((2, 16, _D), jnp.float32),
    ]
    scratch_types = (
        [pltpu.VMEM((2 * _CB,), jnp.int32) for _ in range(2)]
        + [pltpu.VMEM((_CB,), jnp.int32) for _ in range(2)]
        + [pltpu.VMEM((_CB,), jnp.int32) for _ in range(2)]
        + [pltpu.VMEM((_CB,), jnp.int32) for _ in range(2)]
        + [pltpu.VMEM((_CB,), jnp.int32) for _ in range(2)]
        + [pltpu.VMEM((_CB, 64), jnp.float32) for _ in range(2)]
        + [pltpu.VMEM((_CB, 64), jnp.float32) for _ in range(2)]
        + [pltpu.VMEM((_CB, 64), jnp.float32) for _ in range(2)]
        + [pltpu.VMEM((_CB, 64), jnp.float32) for _ in range(2)]
        + [pltpu.SemaphoreType.DMA for _ in range(8)]
        + [
            pltpu.VMEM((128,), jnp.float32),
            pltpu.VMEM_SHARED((_N, 64), jnp.float32),
            pltpu.VMEM_SHARED((_N, 64), jnp.float32),
        ]
    )

    @functools.partial(
        pl.kernel,
        out_type=out_type,
        mesh=mesh,
        scratch_types=scratch_types,
        compiler_params=pltpu.CompilerParams(use_tc_tiling_on_sc=False),
    )
    def k(nm_hbm, ce_hbm, sdi_hbm, enew_hbm, num_hbm, den_hbm, st_hbm, *scr):
        sdiv = scr[0:2]
        bi = scr[2:4]
        di = scr[4:6]
        ei = scr[6:8]
        dsc = scr[8:10]
        bh = scr[10:12]
        dh = scr[12:14]
        eh = scr[14:16]
        cv = scr[16:18]
        sems = scr[18:26]
        statv, nacc, dacc = scr[26:29]

        c = lax.axis_index("c")
        s = lax.axis_index("s")
        zero16 = jnp.zeros((16,), jnp.float32)

        zv = cv[0]

        @pl.loop(0, _CB)
        def _(r):
            for k4 in range(4):
                zv[r, pl.ds(k4 * 16, 16)] = zero16

        for k8 in range(8):
            statv[pl.ds(k8 * 16, 16)] = zero16

        @pl.when(s < 15)
        def _():
            @pl.loop(0, 640 // _CB)
            def _(q):
                row0 = s * 640 + q * _CB
                pltpu.sync_copy(zv, nacc.at[pl.ds(row0, _CB)])
                pltpu.sync_copy(zv, dacc.at[pl.ds(row0, _CB)])

        @pl.when(s == 15)
        def _():
            @pl.loop(0, 400 // _CB)
            def _(q):
                row0 = 9600 + q * _CB
                pltpu.sync_copy(zv, nacc.at[pl.ds(row0, _CB)])
                pltpu.sync_copy(zv, dacc.at[pl.ds(row0, _CB)])

        plsc.subcore_barrier()

        idx_row0 = s * _NCH

        def build(q):
            for j in range(_CB // 16):
                sl = pl.ds(j * 16, 16)
                sh = pl.ds(_CB + j * 16, 16)
                s8 = sdiv[q][sl] * 8
                d0 = sdiv[q][sh]
                bi[q][sl] = s8 + (2 + c)
                di[q][sl] = s8 + (4 + c)
                ei[q][sl] = d0 * 8 + (6 + c)
                dsc[q][sl] = d0

        def fire_in(q, g):
            base = s * _EPT + g * _CB
            sq = sems[4 * q : 4 * q + 4]
            return [
                pltpu.async_copy(nm_hbm.at[bi[q]], bh[q], sq[0]),
                pltpu.async_copy(nm_hbm.at[di[q]], dh[q], sq[1]),
                pltpu.async_copy(nm_hbm.at[ei[q]], eh[q], sq[2]),
                pltpu.async_copy(
                    ce_hbm.at[c].at[pl.ds(base, _CB)], cv[q], sq[3]
                ),
            ]

        def compute(q):
            cvq, dhq, ehq, bhq = cv[q], dh[q], eh[q], bh[q]

            @pl.loop(0, _CB)
            def _(r):
                for k4 in range(4):
                    sl = pl.ds(k4 * 16, 16)
                    en = cvq[r, sl] + dhq[r, sl] + ehq[r, sl]
                    cvq[r, sl] = en
                    sg = 1.0 / (1.0 + jnp.exp(-en))
                    nc = sg * bhq[r, sl]
                    dhq[r, sl] = sg
                    ehq[r, sl] = nc
                    statv[sl] = statv[sl] + en
                    sq = pl.ds(64 + k4 * 16, 16)
                    statv[sq] = statv[sq] + en * en

        def fire_out(q, g):
            base = s * _EPT + g * _CB
            pltpu.sync_copy(cv[q], enew_hbm.at[c].at[pl.ds(base, _CB)])
            pltpu.sync_copy(eh[q], nacc.at[dsc[q]], add=True)
            pltpu.sync_copy(dh[q], dacc.at[dsc[q]], add=True)

        @pl.loop(0, _NCH // 2)
        def _(i):
            ga = 2 * i
            gb = 2 * i + 1
            pltpu.sync_copy(sdi_hbm.at[idx_row0 + ga], sdiv[0])
            build(0)
            h_in_a = fire_in(0, ga)
            pltpu.sync_copy(sdi_hbm.at[idx_row0 + gb], sdiv[1])
            build(1)
            h_in_b = fire_in(1, gb)
            for h in h_in_a:
                h.wait()
            compute(0)
            fire_out(0, ga)
            for h in h_in_b:
                h.wait()
            compute(1)
            fire_out(1, gb)

        plsc.subcore_barrier()

        @pl.when(s < 15)
        def _():
            row0 = s * 640
            pltpu.sync_copy(
                nacc.at[pl.ds(row0, 640)], num_hbm.at[c].at[pl.ds(row0, 640)]
            )
            pltpu.sync_copy(
                dacc.at[pl.ds(row0, 640)], den_hbm.at[c].at[pl.ds(row0, 640)]
            )

        @pl.when(s == 15)
        def _():
            pltpu.sync_copy(
                nacc.at[pl.ds(9600, 400)], num_hbm.at[c].at[pl.ds(9600, 400)]
            )
            pltpu.sync_copy(
                dacc.at[pl.ds(9600, 400)], den_hbm.at[c].at[pl.ds(9600, 400)]
            )

        pltpu.sync_copy(statv, st_hbm.at[c].at[s])

    return k(nm8, ce, sdi)


def _h_update(h_in, ah, num, den, gamma, beta):
    def body(h_ref, ah_ref, num_ref, den_ref, g_ref, b_ref, o_ref):
        num2 = jnp.concatenate([num_ref[0], num_ref[1]], axis=1)
        den2 = jnp.concatenate([den_ref[0], den_ref[1]], axis=1)
        h_new = ah_ref[...] + num2 / (den2 + 1e-6)
        mu = jnp.mean(h_new, axis=0, keepdims=True)
        var = jnp.mean((h_new - mu) ** 2, axis=0, keepdims=True)
        bn = (h_new - mu) * lax.rsqrt(var + 1e-5) * g_ref[...] + b_ref[...]
        o_ref[...] = h_ref[...] + jnp.maximum(bn, 0.0)

    return pl.pallas_call(
        body,
        in_specs=[
            pl.BlockSpec((_N, _D), lambda: (0, 0)),
            pl.BlockSpec((_N, _D), lambda: (0, 0)),
            pl.BlockSpec((2, _N, 64), lambda: (0, 0, 0)),
            pl.BlockSpec((2, _N, 64), lambda: (0, 0, 0)),
            pl.BlockSpec((1, _D), lambda: (0, 0)),
            pl.BlockSpec((1, _D), lambda: (0, 0)),
        ],
        out_specs=pl.BlockSpec((_N, _D), lambda: (0, 0)),
        out_shape=jax.ShapeDtypeStruct((_N, _D), jnp.float32),
    )(h_in, ah, num, den, gamma.reshape(1, _D), beta.reshape(1, _D))


_BEU = 2000


def _e_update(e_in, e_new, scale, shift):
    def body(e_ref, lo_ref, hi_ref, sc_ref, sh_ref, o_ref):
        en = jnp.concatenate([lo_ref[0], hi_ref[0]], axis=1)
        o_ref[...] = e_ref[...] + jnp.maximum(
            en * sc_ref[...] + sh_ref[...], 0.0
        )

    return pl.pallas_call(
        body,
        grid=(_E // _BEU,),
        in_specs=[
            pl.BlockSpec((_BEU, _D), lambda i: (i, 0)),
            pl.BlockSpec((1, _BEU, 64), lambda i: (0, i, 0)),
            pl.BlockSpec((1, _BEU, 64), lambda i: (1, i, 0)),
            pl.BlockSpec((1, _D), lambda i: (0, 0)),
            pl.BlockSpec((1, _D), lambda i: (0, 0)),
        ],
        out_specs=pl.BlockSpec((_BEU, _D), lambda i: (i, 0)),
        out_shape=jax.ShapeDtypeStruct((_E, _D), jnp.float32),
    )(e_in, e_new, e_new, scale, shift)


def _readout(h4, w1, b1, w2, b2, w3p, b3p):
    def body(h_ref, w1_ref, b1_ref, w2_ref, b2_ref, w3_ref, b3_ref, o_ref):
        y = jnp.mean(h_ref[...], axis=0, keepdims=True)
        y = jnp.broadcast_to(y, (8, _D))
        y = jnp.maximum(
            jnp.dot(y, w1_ref[...], preferred_element_type=jnp.float32)
            + b1_ref[...],
            0.0,
        )
        y = jnp.maximum(
            jnp.dot(y, w2_ref[...], preferred_element_type=jnp.float32)
            + b2_ref[...],
            0.0,
        )
        o_ref[...] = (
            jnp.dot(y, w3_ref[...], preferred_element_type=jnp.float32)
            + b3_ref[...]
        )

    return pl.pallas_call(
        body,
        in_specs=[
            pl.BlockSpec((_N, _D), lambda: (0, 0)),
            pl.BlockSpec((_D, _D), lambda: (0, 0)),
            pl.BlockSpec((1, _D), lambda: (0, 0)),
            pl.BlockSpec((_D, _D), lambda: (0, 0)),
            pl.BlockSpec((1, _D), lambda: (0, 0)),
            pl.BlockSpec((_D, _D), lambda: (0, 0)),
            pl.BlockSpec((1, _D), lambda: (0, 0)),
        ],
        out_specs=pl.BlockSpec((8, _D), lambda: (0, 0)),
        out_shape=jax.ShapeDtypeStruct((8, _D), jnp.float32),
    )(
        h4,
        w1,
        b1.reshape(1, _D),
        w2,
        b2.reshape(1, _D),
        w3p,
        b3p.reshape(1, _D),
    )


def kernel(
    h,
    e,
    edge_index,
    W_emb_h,
    b_emb_h,
    W_emb_e,
    b_emb_e,
    W_A,
    b_A,
    W_B,
    b_B,
    W_C,
    b_C,
    W_D,
    b_D,
    W_E,
    b_E,
    gamma_h,
    beta_h,
    gamma_e,
    beta_e,
    W1,
    b1,
    W2,
    b2,
    W3,
    b3,
):
    src = edge_index[0].astype(jnp.int32)
    dst = edge_index[1].astype(jnp.int32)
    sdi = jnp.concatenate(
        [src.reshape(16, _NCH, _CB), dst.reshape(16, _NCH, _CB)], axis=2
    ).reshape(16 * _NCH, 2 * _CB)

    h = _matmul(h, W_emb_h, b_emb_h, 2000)
    e = _matmul(e, W_emb_e, b_emb_e, 2000)

    for l in range(_L):
        wcat = jnp.concatenate([W_A[l], W_B[l], W_D[l], W_E[l]], axis=1)
        bcat = jnp.concatenate([b_A[l], b_B[l], b_D[l], b_E[l]], axis=0)
        nm = _matmul(h, wcat, bcat, 2000)
        ce = _matmul_split(e, W_C[l], b_C[l], 2000)
        e_new, num, den, st = _sc_edge(nm.reshape(8 * _N, 64), ce, sdi)
        ah = lax.slice(nm, (0, 0), (_N, _D))
        h = _h_update(h, ah, num, den, gamma_h[l], beta_h[l])
        if l < _L - 1:
            cnt = float(_E)
            ssum = jnp.concatenate(
                [st[0, :, :64].sum(axis=0), st[1, :, :64].sum(axis=0)]
            )
            ssq = jnp.concatenate(
                [st[0, :, 64:].sum(axis=0), st[1, :, 64:].sum(axis=0)]
            )
            mu = ssum / cnt
            var = ssq / cnt - mu * mu
            rstd = lax.rsqrt(var + 1e-5)
            scale = (gamma_e[l] * rstd).reshape(1, _D)
            shift = (beta_e[l] - mu * rstd * gamma_e[l]).reshape(1, _D)
            e = _e_update(e, e_new, scale, shift)

    w3p = jnp.zeros((_D, _D), jnp.float32).at[:, :10].set(W3)
    b3p = jnp.zeros((_D,), jnp.float32).at[:10].set(b3)
    y = _readout(h, W1, b1, W2, b2, w3p, b3p)
    return y[0:1, 0:10]


# speedup vs baseline: 1.2552x; 1.0351x over previous
import functools

import jax
import jax.numpy as jnp
from jax import lax
from jax.experimental import pallas as pl
from jax.experimental.pallas import tpu as pltpu
from jax.experimental.pallas import tpu_sc as plsc

_N = 10000
_E = 320000
_D = 128
_L = 4


def _mm_body(x_ref, w_ref, b_ref, o_ref):
    o_ref[...] = (
        jnp.dot(x_ref[...], w_ref[...], preferred_element_type=jnp.float32)
        + b_ref[...]
    )


def _matmul(x, w, b, block_rows):
    rows, k = x.shape
    dout = w.shape[1]
    return pl.pallas_call(
        _mm_body,
        grid=(rows // block_rows,),
        in_specs=[
            pl.BlockSpec((block_rows, k), lambda i: (i, 0)),
            pl.BlockSpec((k, dout), lambda i: (0, 0)),
            pl.BlockSpec((1, dout), lambda i: (0, 0)),
        ],
        out_specs=pl.BlockSpec((block_rows, dout), lambda i: (i, 0)),
        out_shape=jax.ShapeDtypeStruct((rows, dout), jnp.float32),
    )(x, w, b.reshape(1, dout))


def _mm_split_body(x_ref, w_ref, b_ref, o_ref):
    o_ref[0] = (
        jnp.dot(x_ref[...], w_ref[0], preferred_element_type=jnp.float32)
        + b_ref[0]
    )


def _matmul_split(x, w, b, block_rows):
    rows, k = x.shape
    wsp = w.reshape(k, 2, 64).transpose(1, 0, 2)
    bsp = b.reshape(2, 1, 64)
    return pl.pallas_call(
        _mm_split_body,
        grid=(rows // block_rows, 2),
        in_specs=[
            pl.BlockSpec((block_rows, k), lambda i, c: (i, 0)),
            pl.BlockSpec((1, k, 64), lambda i, c: (c, 0, 0)),
            pl.BlockSpec((1, 1, 64), lambda i, c: (c, 0, 0)),
        ],
        out_specs=pl.BlockSpec((1, block_rows, 64), lambda i, c: (c, i, 0)),
        out_shape=jax.ShapeDtypeStruct((2, rows, 64), jnp.float32),
    )(x, wsp, bsp)


_CB = 80
_EPT = _E // 16
_NCH = _EPT // _CB


def _sc_edge(nm8, ce, sdi):
    mesh = plsc.VectorSubcoreMesh(core_axis_name="c", subcore_axis_name="s")
    out_type = [
        jax.ShapeDtypeStruct((2, _E, 64), jnp.float32),
        jax.ShapeDtypeStruct((2, _N, 64), jnp.float32),
        jax.ShapeDtypeStruct((2, _N, 64), jnp.float32),
        jax.ShapeDtypeStruct((2, 16, _D), jnp.float32),
    ]
    scratch_types = (
        [pltpu.VMEM((2 * _CB,), jnp.int32) for _ in range(2)]
        + [pltpu.VMEM((_CB,), jnp.int32) for _ in range(2)]
        + [pltpu.VMEM((_CB,), jnp.int32) for _ in range(2)]
        + [pltpu.VMEM((_CB,), jnp.int32) for _ in range(2)]
        + [pltpu.VMEM((_CB,), jnp.int32) for _ in range(2)]
        + [pltpu.VMEM((_CB, 64), jnp.float32) for _ in range(2)]
        + [pltpu.VMEM((_CB, 64), jnp.float32) for _ in range(2)]
        + [pltpu.VMEM((_CB, 64), jnp.float32) for _ in range(2)]
        + [pltpu.VMEM((_CB, 64), jnp.float32) for _ in range(2)]
        + [pltpu.SemaphoreType.DMA for _ in range(14)]
        + [
            pltpu.VMEM((128,), jnp.float32),
            pltpu.VMEM_SHARED((_N, 64), jnp.float32),
            pltpu.VMEM_SHARED((_N, 64), jnp.float32),
        ]
    )

    @functools.partial(
        pl.kernel,
        out_type=out_type,
        mesh=mesh,
        scratch_types=scratch_types,
        compiler_params=pltpu.CompilerParams(use_tc_tiling_on_sc=False),
    )
    def k(nm_hbm, ce_hbm, sdi_hbm, enew_hbm, num_hbm, den_hbm, st_hbm, *scr):
        sdiv = scr[0:2]
        bi = scr[2:4]
        di = scr[4:6]
        ei = scr[6:8]
        dsc = scr[8:10]
        bh = scr[10:12]
        dh = scr[12:14]
        eh = scr[14:16]
        cv = scr[16:18]
        sems = scr[18:26]
        osems = scr[26:32]
        statv, nacc, dacc = scr[32:35]

        c = lax.axis_index("c")
        s = lax.axis_index("s")
        zero16 = jnp.zeros((16,), jnp.float32)

        zv = cv[0]

        @pl.loop(0, _CB)
        def _(r):
            for k4 in range(4):
                zv[r, pl.ds(k4 * 16, 16)] = zero16

        for k8 in range(8):
            statv[pl.ds(k8 * 16, 16)] = zero16

        @pl.when(s < 15)
        def _():
            @pl.loop(0, 640 // _CB)
            def _(q):
                row0 = s * 640 + q * _CB
                pltpu.sync_copy(zv, nacc.at[pl.ds(row0, _CB)])
                pltpu.sync_copy(zv, dacc.at[pl.ds(row0, _CB)])

        @pl.when(s == 15)
        def _():
            @pl.loop(0, 400 // _CB)
            def _(q):
                row0 = 9600 + q * _CB
                pltpu.sync_copy(zv, nacc.at[pl.ds(row0, _CB)])
                pltpu.sync_copy(zv, dacc.at[pl.ds(row0, _CB)])

        plsc.subcore_barrier()

        idx_row0 = s * _NCH

        def build(q):
            for j in range(_CB // 16):
                sl = pl.ds(j * 16, 16)
                sh = pl.ds(_CB + j * 16, 16)
                s8 = sdiv[q][sl] * 8
                d0 = sdiv[q][sh]
                bi[q][sl] = s8 + (2 + c)
                di[q][sl] = s8 + (4 + c)
                ei[q][sl] = d0 * 8 + (6 + c)
                dsc[q][sl] = d0

        def fire_in(q, g):
            base = s * _EPT + g * _CB
            sq = sems[4 * q : 4 * q + 4]
            return [
                pltpu.async_copy(nm_hbm.at[bi[q]], bh[q], sq[0]),
                pltpu.async_copy(nm_hbm.at[di[q]], dh[q], sq[1]),
                pltpu.async_copy(nm_hbm.at[ei[q]], eh[q], sq[2]),
                pltpu.async_copy(
                    ce_hbm.at[c].at[pl.ds(base, _CB)], cv[q], sq[3]
                ),
            ]

        def compute(q):
            cvq, dhq, ehq, bhq = cv[q], dh[q], eh[q], bh[q]

            @pl.loop(0, _CB)
            def _(r):
                for k4 in range(4):
                    sl = pl.ds(k4 * 16, 16)
                    en = cvq[r, sl] + dhq[r, sl] + ehq[r, sl]
                    cvq[r, sl] = en
                    sg = 1.0 / (1.0 + jnp.exp(-en))
                    nc = sg * bhq[r, sl]
                    dhq[r, sl] = sg
                    ehq[r, sl] = nc
                    statv[sl] = statv[sl] + en
                    sq = pl.ds(64 + k4 * 16, 16)
                    statv[sq] = statv[sq] + en * en

        def fire_out(q, g):
            base = s * _EPT + g * _CB
            sq = osems[3 * q : 3 * q + 3]
            return [
                pltpu.async_copy(
                    cv[q], enew_hbm.at[c].at[pl.ds(base, _CB)], sq[0]
                ),
                pltpu.async_copy(eh[q], nacc.at[dsc[q]], sq[1], add=True),
                pltpu.async_copy(dh[q], dacc.at[dsc[q]], sq[2], add=True),
            ]

        @pl.loop(0, _NCH // 2)
        def _(i):
            ga = 2 * i
            gb = 2 * i + 1
            pltpu.sync_copy(sdi_hbm.at[idx_row0 + ga], sdiv[0])
            build(0)
            h_in_a = fire_in(0, ga)
            pltpu.sync_copy(sdi_hbm.at[idx_row0 + gb], sdiv[1])
            build(1)
            h_in_b = fire_in(1, gb)
            for h in h_in_a:
                h.wait()
            compute(0)
            h_out_a = fire_out(0, ga)
            for h in h_in_b:
                h.wait()
            compute(1)
            h_out_b = fire_out(1, gb)
            for h in h_out_a + h_out_b:
                h.wait()

        plsc.subcore_barrier()

        @pl.when(s < 15)
        def _():
            row0 = s * 640
            pltpu.sync_copy(
                nacc.at[pl.ds(row0, 640)], num_hbm.at[c].at[pl.ds(row0, 640)]
            )
            pltpu.sync_copy(
                dacc.at[pl.ds(row0, 640)], den_hbm.at[c].at[pl.ds(row0, 640)]
            )

        @pl.when(s == 15)
        def _():
            pltpu.sync_copy(
                nacc.at[pl.ds(9600, 400)], num_hbm.at[c].at[pl.ds(9600, 400)]
            )
            pltpu.sync_copy(
                dacc.at[pl.ds(9600, 400)], den_hbm.at[c].at[pl.ds(9600, 400)]
            )

        pltpu.sync_copy(statv, st_hbm.at[c].at[s])

    return k(nm8, ce, sdi)


def _h_update(h_in, ah, num, den, gamma, beta):
    def body(h_ref, ah_ref, num_ref, den_ref, g_ref, b_ref, o_ref):
        num2 = jnp.concatenate([num_ref[0], num_ref[1]], axis=1)
        den2 = jnp.concatenate([den_ref[0], den_ref[1]], axis=1)
        h_new = ah_ref[...] + num2 / (den2 + 1e-6)
        mu = jnp.mean(h_new, axis=0, keepdims=True)
        var = jnp.mean((h_new - mu) ** 2, axis=0, keepdims=True)
        bn = (h_new - mu) * lax.rsqrt(var + 1e-5) * g_ref[...] + b_ref[...]
        o_ref[...] = h_ref[...] + jnp.maximum(bn, 0.0)

    return pl.pallas_call(
        body,
        in_specs=[
            pl.BlockSpec((_N, _D), lambda: (0, 0)),
            pl.BlockSpec((_N, _D), lambda: (0, 0)),
            pl.BlockSpec((2, _N, 64), lambda: (0, 0, 0)),
            pl.BlockSpec((2, _N, 64), lambda: (0, 0, 0)),
            pl.BlockSpec((1, _D), lambda: (0, 0)),
            pl.BlockSpec((1, _D), lambda: (0, 0)),
        ],
        out_specs=pl.BlockSpec((_N, _D), lambda: (0, 0)),
        out_shape=jax.ShapeDtypeStruct((_N, _D), jnp.float32),
    )(h_in, ah, num, den, gamma.reshape(1, _D), beta.reshape(1, _D))


_BEU = 2000


def _e_update(e_in, e_new, scale, shift):
    def body(e_ref, lo_ref, hi_ref, sc_ref, sh_ref, o_ref):
        en = jnp.concatenate([lo_ref[0], hi_ref[0]], axis=1)
        o_ref[...] = e_ref[...] + jnp.maximum(
            en * sc_ref[...] + sh_ref[...], 0.0
        )

    return pl.pallas_call(
        body,
        grid=(_E // _BEU,),
        in_specs=[
            pl.BlockSpec((_BEU, _D), lambda i: (i, 0)),
            pl.BlockSpec((1, _BEU, 64), lambda i: (0, i, 0)),
            pl.BlockSpec((1, _BEU, 64), lambda i: (1, i, 0)),
            pl.BlockSpec((1, _D), lambda i: (0, 0)),
            pl.BlockSpec((1, _D), lambda i: (0, 0)),
        ],
        out_specs=pl.BlockSpec((_BEU, _D), lambda i: (i, 0)),
        out_shape=jax.ShapeDtypeStruct((_E, _D), jnp.float32),
    )(e_in, e_new, e_new, scale, shift)


def _readout(h4, w1, b1, w2, b2, w3p, b3p):
    def body(h_ref, w1_ref, b1_ref, w2_ref, b2_ref, w3_ref, b3_ref, o_ref):
        y = jnp.mean(h_ref[...], axis=0, keepdims=True)
        y = jnp.broadcast_to(y, (8, _D))
        y = jnp.maximum(
            jnp.dot(y, w1_ref[...], preferred_element_type=jnp.float32)
            + b1_ref[...],
            0.0,
        )
        y = jnp.maximum(
            jnp.dot(y, w2_ref[...], preferred_element_type=jnp.float32)
            + b2_ref[...],
            0.0,
        )
        o_ref[...] = (
            jnp.dot(y, w3_ref[...], preferred_element_type=jnp.float32)
            + b3_ref[...]
        )

    return pl.pallas_call(
        body,
        in_specs=[
            pl.BlockSpec((_N, _D), lambda: (0, 0)),
            pl.BlockSpec((_D, _D), lambda: (0, 0)),
            pl.BlockSpec((1, _D), lambda: (0, 0)),
            pl.BlockSpec((_D, _D), lambda: (0, 0)),
            pl.BlockSpec((1, _D), lambda: (0, 0)),
            pl.BlockSpec((_D, _D), lambda: (0, 0)),
            pl.BlockSpec((1, _D), lambda: (0, 0)),
        ],
        out_specs=pl.BlockSpec((8, _D), lambda: (0, 0)),
        out_shape=jax.ShapeDtypeStruct((8, _D), jnp.float32),
    )(
        h4,
        w1,
        b1.reshape(1, _D),
        w2,
        b2.reshape(1, _D),
        w3p,
        b3p.reshape(1, _D),
    )


def kernel(
    h,
    e,
    edge_index,
    W_emb_h,
    b_emb_h,
    W_emb_e,
    b_emb_e,
    W_A,
    b_A,
    W_B,
    b_B,
    W_C,
    b_C,
    W_D,
    b_D,
    W_E,
    b_E,
    gamma_h,
    beta_h,
    gamma_e,
    beta_e,
    W1,
    b1,
    W2,
    b2,
    W3,
    b3,
):
    src = edge_index[0].astype(jnp.int32)
    dst = edge_index[1].astype(jnp.int32)
    sdi = jnp.concatenate(
        [src.reshape(16, _NCH, _CB), dst.reshape(16, _NCH, _CB)], axis=2
    ).reshape(16 * _NCH, 2 * _CB)

    h = _matmul(h, W_emb_h, b_emb_h, 2000)
    e = _matmul(e, W_emb_e, b_emb_e, 2000)

    for l in range(_L):
        wcat = jnp.concatenate([W_A[l], W_B[l], W_D[l], W_E[l]], axis=1)
        bcat = jnp.concatenate([b_A[l], b_B[l], b_D[l], b_E[l]], axis=0)
        nm = _matmul(h, wcat, bcat, 2000)
        ce = _matmul_split(e, W_C[l], b_C[l], 2000)
        e_new, num, den, st = _sc_edge(nm.reshape(8 * _N, 64), ce, sdi)
        ah = lax.slice(nm, (0, 0), (_N, _D))
        h = _h_update(h, ah, num, den, gamma_h[l], beta_h[l])
        if l < _L - 1:
            cnt = float(_E)
            ssum = jnp.concatenate(
                [st[0, :, :64].sum(axis=0), st[1, :, :64].sum(axis=0)]
            )
            ssq = jnp.concatenate(
                [st[0, :, 64:].sum(axis=0), st[1, :, 64:].sum(axis=0)]
            )
            mu = ssum / cnt
            var = ssq / cnt - mu * mu
            rstd = lax.rsqrt(var + 1e-5)
            scale = (gamma_e[l] * rstd).reshape(1, _D)
            shift = (beta_e[l] - mu * rstd * gamma_e[l]).reshape(1, _D)
            e = _e_update(e, e_new, scale, shift)

    w3p = jnp.zeros((_D, _D), jnp.float32).at[:, :10].set(W3)
    b3p = jnp.zeros((_D,), jnp.float32).at[:10].set(b3)
    y = _readout(h, W1, b1, W2, b2, w3p, b3p)
    return y[0:1, 0:10]
